# Initial kernel scaffold; baseline (speedup 1.0000x reference)
import jax
import jax.numpy as jnp
from jax.experimental import pallas as pl


def kernel(x, W, b_conv):
    raise NotImplementedError("write your pallas kernel here")


# speedup vs baseline: 6172.6492x; 6172.6492x over previous
import functools

import jax
import jax.numpy as jnp
import numpy as np
from jax import lax
from jax.experimental import pallas as pl
from jax.experimental.pallas import tpu as pltpu
from jax.experimental.pallas import tpu_sc as plsc

C = 96
H = 224
W_DIM = 224
PAD_T = 2
PAD_L = 3
HP = H + 2 * PAD_T
WP = W_DIM + 2 * PAD_L
N_ALL = 20
N_SEL = 16
TH = 8
N_TILES_1 = H // TH
TP = 896
N_TILES_3 = (H * W_DIM) // TP
PIX = H * W_DIM
ROWS = N_SEL * PIX
N_WORKERS = 32
ROWS_PER_W = ROWS // N_WORKERS
CHUNK = 128
N_CHUNKS = ROWS_PER_W // CHUNK
CP = 128


def _ring_offsets():
    rb, gbh, gbw = 1, 1, 2
    h_prf = (rb + gbh) * 2 + 1
    w_prf = (rb + gbw) * 2 + 1
    xs = np.arange(-(rb + gbh), rb + gbh + 1)
    ys = np.arange(-(rb + gbw), rb + gbw + 1)
    gx, gy = np.meshgrid(xs, ys, indexing="ij")

    def ring(a):
        t = a[0:rb].ravel()
        r = a[rb:h_prf - rb, w_prf - rb:w_prf].ravel()
        d = a[h_prf - rb:h_prf].ravel()
        l = a[rb:h_prf - rb, 0:rb].ravel()
        return np.concatenate([t, r, d, l], 0)

    return list(zip(ring(gx).tolist(), ring(gy).tolist()))


OFFS = _ring_offsets()


def _stage1_body(xh_ref, idx_ref):
    i = pl.program_id(0)
    xc = xh_ref[0, :, PAD_T:PAD_T + TH, PAD_L:PAD_L + W_DIM]
    s = []
    for (dh, dw) in OFFS:
        xs = xh_ref[0, :, PAD_T + dh:PAD_T + dh + TH, PAD_L + dw:PAD_L + dw + W_DIM]
        s.append(jnp.sum(xs * xc, axis=0))
    ranks = []
    for n in range(N_ALL):
        acc = jnp.zeros((TH, W_DIM), jnp.int32)
        for m in range(N_ALL):
            if m == n:
                continue
            lt = s[m] < s[n]
            if m < n:
                lt = lt | (s[m] == s[n])
            acc = acc + lt.astype(jnp.int32)
        ranks.append(acc)
    h0 = i * TH
    row = h0 + lax.broadcasted_iota(jnp.int32, (TH, W_DIM), 0)
    col = lax.broadcasted_iota(jnp.int32, (TH, W_DIM), 1)
    base = (row + PAD_T) * WP + (col + PAD_L)
    for k in range(N_SEL):
        acc = jnp.zeros((TH, W_DIM), jnp.int32)
        for n, (dh, dw) in enumerate(OFFS):
            flat = base + (dh * WP + dw)
            acc = acc + jnp.where(ranks[n] == k, flat, 0)
        idx_ref[k] = acc


def _stage1(x_halo):
    return pl.pallas_call(
        _stage1_body,
        grid=(N_TILES_1,),
        in_specs=[pl.BlockSpec((1, C, TH + 2 * PAD_T, WP), lambda i: (i, 0, 0, 0))],
        out_specs=pl.BlockSpec((N_SEL, TH, W_DIM), lambda i: (0, i, 0)),
        out_shape=jax.ShapeDtypeStruct((N_SEL, H, W_DIM), jnp.int32),
    )(x_halo)


def _sc_gather(table, idx_flat):
    mesh = plsc.VectorSubcoreMesh(core_axis_name="c", subcore_axis_name="s")

    @functools.partial(
        pl.kernel,
        mesh=mesh,
        out_type=jax.ShapeDtypeStruct((ROWS, CP), jnp.float32),
        scratch_types=[
            pltpu.VMEM((CHUNK,), jnp.int32),
            pltpu.VMEM((CHUNK, CP), jnp.float32),
            pltpu.SemaphoreType.DMA,
        ],
    )
    def k(table_hbm, idx_hbm, y_hbm, idx_v, rows_v, sem):
        wid = lax.axis_index("s") * 2 + lax.axis_index("c")
        base = wid * ROWS_PER_W

        def body(j, carry):
            off = base + j * CHUNK
            pltpu.sync_copy(idx_hbm.at[pl.ds(off, CHUNK)], idx_v)
            pltpu.async_copy(table_hbm.at[idx_v], rows_v, sem).wait()
            pltpu.sync_copy(rows_v, y_hbm.at[pl.ds(off, CHUNK)])
            return carry

        lax.fori_loop(0, N_CHUNKS, body, 0)

    return k(table, idx_flat)


def _stage3_body(wk_ref, wsum_ref, b_ref, x_ref, y_ref, out_ref):
    acc = lax.dot_general(
        wsum_ref[...], x_ref[...], (((1,), (0,)), ((), ())),
        preferred_element_type=jnp.float32, precision=lax.Precision.HIGHEST)
    for k in range(N_SEL):
        acc = acc - lax.dot_general(
            wk_ref[k], y_ref[k], (((1,), (1,)), ((), ())),
            preferred_element_type=jnp.float32, precision=lax.Precision.HIGHEST)
    out_ref[...] = acc + b_ref[...]


def _stage3(wk, wsum, b2, x_cm, y):
    return pl.pallas_call(
        _stage3_body,
        grid=(N_TILES_3,),
        in_specs=[
            pl.BlockSpec((N_SEL, C, CP), lambda i: (0, 0, 0)),
            pl.BlockSpec((C, C), lambda i: (0, 0)),
            pl.BlockSpec((C, 1), lambda i: (0, 0)),
            pl.BlockSpec((C, TP), lambda i: (0, i)),
            pl.BlockSpec((N_SEL, TP, CP), lambda i: (0, i, 0)),
        ],
        out_specs=pl.BlockSpec((C, TP), lambda i: (0, i)),
        out_shape=jax.ShapeDtypeStruct((C, PIX), jnp.float32),
    )(wk, wsum, b2, x_cm, y)


def kernel(x, W, b_conv):
    xsq = x[0]
    x_pad = jnp.pad(xsq, ((0, 0), (PAD_T, PAD_T), (PAD_L, PAD_L)))
    x_halo = jnp.stack(
        [x_pad[:, TH * i:TH * i + TH + 2 * PAD_T, :] for i in range(N_TILES_1)])
    idx = _stage1(x_halo)
    table = jnp.pad(
        jnp.transpose(x_pad.reshape(C, HP * WP), (1, 0)), ((0, 0), (0, CP - C)))
    y = _sc_gather(table, idx.reshape(-1))
    w2 = W.reshape(C, C, N_SEL)
    wk = jnp.pad(jnp.transpose(w2, (2, 0, 1)), ((0, 0), (0, 0), (0, CP - C)))
    wsum = jnp.sum(w2, axis=2)
    x_cm = xsq.reshape(C, PIX)
    out = _stage3(wk, wsum, b_conv.reshape(C, 1), x_cm, y.reshape(N_SEL, PIX, CP))
    return out.reshape(1, C, H, W_DIM)


# speedup vs baseline: 14303.6944x; 2.3173x over previous
import functools

import jax
import jax.numpy as jnp
import numpy as np
from jax import lax
from jax.experimental import pallas as pl
from jax.experimental.pallas import tpu as pltpu
from jax.experimental.pallas import tpu_sc as plsc

C = 96
H = 224
W_DIM = 224
PAD_T = 2
PAD_L = 3
HP = H + 2 * PAD_T
WP = W_DIM + 2 * PAD_L
N_ALL = 20
N_SEL = 16
CG = 2
TH = 8
N_TILES_1 = H // TH
TP = 896
N_TILES_3 = (H * W_DIM) // TP
PIX = H * W_DIM
ROWS = N_SEL * PIX
N_WORKERS = 32
ROWS_PER_W = ROWS // N_WORKERS
CHUNK = 128
N_CHUNKS = ROWS_PER_W // CHUNK
NBUF = 4
N_GROUPS = N_CHUNKS // NBUF
CP = 128


def _ring_offsets():
    rb, gbh, gbw = 1, 1, 2
    h_prf = (rb + gbh) * 2 + 1
    w_prf = (rb + gbw) * 2 + 1
    xs = np.arange(-(rb + gbh), rb + gbh + 1)
    ys = np.arange(-(rb + gbw), rb + gbw + 1)
    gx, gy = np.meshgrid(xs, ys, indexing="ij")

    def ring(a):
        t = a[0:rb].ravel()
        r = a[rb:h_prf - rb, w_prf - rb:w_prf].ravel()
        d = a[h_prf - rb:h_prf].ravel()
        l = a[rb:h_prf - rb, 0:rb].ravel()
        return np.concatenate([t, r, d, l], 0)

    return list(zip(ring(gx).tolist(), ring(gy).tolist()))


OFFS = _ring_offsets()


def _stage1_body(xp_ref, idx_ref):
    i = pl.program_id(0)
    h0 = i * TH
    s = [jnp.zeros((TH, W_DIM), jnp.float32) for _ in range(N_ALL)]
    for g in range(C // CG):
        xw = xp_ref[pl.ds(g * CG, CG), pl.ds(h0, TH + 8), :]
        xcg = xw[:, PAD_T:PAD_T + TH, PAD_L:PAD_L + W_DIM]
        for n, (dh, dw) in enumerate(OFFS):
            xsg = xw[:, PAD_T + dh:PAD_T + dh + TH,
                     PAD_L + dw:PAD_L + dw + W_DIM]
            s[n] = s[n] + jnp.sum(xcg * xsg, axis=0)
    ranks = []
    for n in range(N_ALL):
        acc = jnp.zeros((TH, W_DIM), jnp.int32)
        for m in range(N_ALL):
            if m == n:
                continue
            lt = s[m] < s[n]
            if m < n:
                lt = lt | (s[m] == s[n])
            acc = acc + lt.astype(jnp.int32)
        ranks.append(acc)
    row = h0 + lax.broadcasted_iota(jnp.int32, (TH, W_DIM), 0)
    col = lax.broadcasted_iota(jnp.int32, (TH, W_DIM), 1)
    base = (row + PAD_T) * WP + (col + PAD_L)
    for k in range(N_SEL):
        acc = jnp.zeros((TH, W_DIM), jnp.int32)
        for n in range(N_ALL):
            dh, dw = OFFS[n]
            flat = base + (dh * WP + dw)
            acc = acc + jnp.where(ranks[n] == k, flat, 0)
        idx_ref[k] = acc


def _stage1(x_pad):
    return pl.pallas_call(
        _stage1_body,
        grid=(N_TILES_1,),
        in_specs=[pl.BlockSpec((C, HP + 4, WP), lambda i: (0, 0, 0))],
        out_specs=pl.BlockSpec((N_SEL, TH, W_DIM), lambda i: (0, i, 0)),
        out_shape=jax.ShapeDtypeStruct((N_SEL, H, W_DIM), jnp.int32),
    )(x_pad)


def _sc_gather(table, idx_flat):
    mesh = plsc.VectorSubcoreMesh(core_axis_name="c", subcore_axis_name="s")

    @functools.partial(
        pl.kernel,
        mesh=mesh,
        out_type=jax.ShapeDtypeStruct((ROWS, CP), jnp.float32),
        scratch_types=[
            pltpu.VMEM((ROWS_PER_W,), jnp.int32),
            pltpu.VMEM((NBUF, CHUNK, CP), jnp.float32),
            pltpu.SemaphoreType.DMA,
            pltpu.SemaphoreType.DMA,
        ],
    )
    def k(table_hbm, idx_hbm, y_hbm, idx_all, rows, gsem, wsem):
        wid = lax.axis_index("s") * 2 + lax.axis_index("c")
        base = wid * ROWS_PER_W
        pltpu.sync_copy(idx_hbm.at[pl.ds(base, ROWS_PER_W)], idx_all)

        def gather(j, b):
            pltpu.async_copy(
                table_hbm.at[idx_all.at[pl.ds(j * CHUNK, CHUNK)]],
                rows.at[b], gsem)

        def gwait(b):
            pltpu.make_async_copy(
                table_hbm.at[pl.ds(0, CHUNK)], rows.at[b], gsem).wait()

        def wback(j, b):
            pltpu.async_copy(
                rows.at[b], y_hbm.at[pl.ds(base + j * CHUNK, CHUNK)], wsem)

        def wwait(b):
            pltpu.make_async_copy(
                table_hbm.at[pl.ds(0, CHUNK)], rows.at[b], wsem).wait()

        for b in range(NBUF):
            gather(b, b)

        def body(j0, carry):
            for b in range(NBUF):
                gwait(b)
                wback(j0 * NBUF + b, b)
            for b in range(NBUF):
                nxt = (j0 + 1) * NBUF + b

                @pl.when(nxt < N_CHUNKS)
                def _():
                    wwait(b)
                    gather(nxt, b)

            return carry

        lax.fori_loop(0, N_GROUPS, body, 0)
        for b in range(NBUF):
            wwait(b)

    return k(table, idx_flat)


def _stage3_body(wk_ref, wsum_ref, b_ref, x_ref, y_ref, out_ref):
    acc = lax.dot_general(
        wsum_ref[...], x_ref[...], (((1,), (0,)), ((), ())),
        preferred_element_type=jnp.float32, precision=lax.Precision.DEFAULT)
    for k in range(N_SEL):
        acc = acc - lax.dot_general(
            wk_ref[k], y_ref[k], (((1,), (1,)), ((), ())),
            preferred_element_type=jnp.float32, precision=lax.Precision.DEFAULT)
    out_ref[...] = acc + b_ref[...]


def _stage3(wk, wsum, b2, x_cm, y):
    return pl.pallas_call(
        _stage3_body,
        grid=(N_TILES_3,),
        in_specs=[
            pl.BlockSpec((N_SEL, C, CP), lambda i: (0, 0, 0)),
            pl.BlockSpec((C, C), lambda i: (0, 0)),
            pl.BlockSpec((C, 1), lambda i: (0, 0)),
            pl.BlockSpec((C, TP), lambda i: (0, i)),
            pl.BlockSpec((N_SEL, TP, CP), lambda i: (0, i, 0)),
        ],
        out_specs=pl.BlockSpec((C, TP), lambda i: (0, i)),
        out_shape=jax.ShapeDtypeStruct((C, PIX), jnp.float32),
    )(wk, wsum, b2, x_cm, y)


def kernel(x, W, b_conv):
    xsq = x[0]
    x_pad4 = jnp.pad(xsq, ((0, 0), (PAD_T, PAD_T + 4), (PAD_L, PAD_L)))
    idx = _stage1(x_pad4)
    x_pad = x_pad4[:, :HP, :]
    table = jnp.pad(
        jnp.transpose(x_pad.reshape(C, HP * WP), (1, 0)), ((0, 0), (0, CP - C)))
    y = _sc_gather(table, idx.reshape(-1))
    w2 = W.reshape(C, C, N_SEL)
    wk = jnp.pad(jnp.transpose(w2, (2, 0, 1)), ((0, 0), (0, 0), (0, CP - C)))
    wsum = jnp.sum(w2, axis=2)
    x_cm = xsq.reshape(C, PIX)
    out = _stage3(wk, wsum, b_conv.reshape(C, 1), x_cm, y.reshape(N_SEL, PIX, CP))
    return out.reshape(1, C, H, W_DIM)


# speedup vs baseline: 15840.0114x; 1.1074x over previous
import functools

import jax
import jax.numpy as jnp
import numpy as np
from jax import lax
from jax.experimental import pallas as pl
from jax.experimental.pallas import tpu as pltpu
from jax.experimental.pallas import tpu_sc as plsc

C = 96
H = 224
W_DIM = 224
PAD_T = 2
PAD_L = 3
HP = H + 2 * PAD_T
WP = W_DIM + 2 * PAD_L
N_ALL = 20
N_SEL = 16
CG = 2
TH = 8
HALF = 2
H_HALF = H // HALF
N_TILES_1 = H_HALF // TH
PIX_H = H_HALF * W_DIM
TP = 896
N_TILES_3 = PIX_H // TP
ROWS_H = N_SEL * PIX_H
N_WORKERS = 32
ROWS_PER_W = ROWS_H // N_WORKERS
CHUNK = 128
N_CHUNKS = ROWS_PER_W // CHUNK
NBUF = 2
N_GROUPS = N_CHUNKS // NBUF
CP = 128


def _ring_offsets():
    rb, gbh, gbw = 1, 1, 2
    h_prf = (rb + gbh) * 2 + 1
    w_prf = (rb + gbw) * 2 + 1
    xs = np.arange(-(rb + gbh), rb + gbh + 1)
    ys = np.arange(-(rb + gbw), rb + gbw + 1)
    gx, gy = np.meshgrid(xs, ys, indexing="ij")

    def ring(a):
        t = a[0:rb].ravel()
        r = a[rb:h_prf - rb, w_prf - rb:w_prf].ravel()
        d = a[h_prf - rb:h_prf].ravel()
        l = a[rb:h_prf - rb, 0:rb].ravel()
        return np.concatenate([t, r, d, l], 0)

    return list(zip(ring(gx).tolist(), ring(gy).tolist()))


OFFS = _ring_offsets()


def _stage1_body(h_base, xp_ref, idx_ref):
    i = pl.program_id(0)
    h0 = h_base + i * TH
    s = [jnp.zeros((TH, W_DIM), jnp.float32) for _ in range(N_ALL)]
    for g in range(C // CG):
        xw = xp_ref[pl.ds(g * CG, CG), pl.ds(h0, TH + 8), :]
        xcg = xw[:, PAD_T:PAD_T + TH, PAD_L:PAD_L + W_DIM]
        for n, (dh, dw) in enumerate(OFFS):
            xsg = xw[:, PAD_T + dh:PAD_T + dh + TH,
                     PAD_L + dw:PAD_L + dw + W_DIM]
            s[n] = s[n] + jnp.sum(xcg * xsg, axis=0)
    ranks = []
    for n in range(N_ALL):
        acc = jnp.zeros((TH, W_DIM), jnp.int32)
        for m in range(N_ALL):
            if m == n:
                continue
            lt = s[m] < s[n]
            if m < n:
                lt = lt | (s[m] == s[n])
            acc = acc + lt.astype(jnp.int32)
        ranks.append(acc)
    row = h0 + lax.broadcasted_iota(jnp.int32, (TH, W_DIM), 0)
    col = lax.broadcasted_iota(jnp.int32, (TH, W_DIM), 1)
    base = (row + PAD_T) * WP + (col + PAD_L)
    for k in range(N_SEL):
        acc = jnp.zeros((TH, W_DIM), jnp.int32)
        for n in range(N_ALL):
            dh, dw = OFFS[n]
            flat = base + (dh * WP + dw)
            acc = acc + jnp.where(ranks[n] == k, flat, 0)
        idx_ref[k] = acc


def _stage1(x_pad, h_base):
    return pl.pallas_call(
        functools.partial(_stage1_body, h_base),
        grid=(N_TILES_1,),
        in_specs=[pl.BlockSpec((C, HP + 4, WP), lambda i: (0, 0, 0))],
        out_specs=pl.BlockSpec((N_SEL, TH, W_DIM), lambda i: (0, i, 0)),
        out_shape=jax.ShapeDtypeStruct((N_SEL, H_HALF, W_DIM), jnp.int32),
    )(x_pad)


def _sc_gather(table, idx_flat):
    mesh = plsc.VectorSubcoreMesh(core_axis_name="c", subcore_axis_name="s")

    @functools.partial(
        pl.kernel,
        mesh=mesh,
        out_type=jax.ShapeDtypeStruct((ROWS_H, CP), jnp.float32),
        scratch_types=[
            pltpu.VMEM((ROWS_PER_W,), jnp.int32),
            pltpu.VMEM((NBUF, CHUNK, CP), jnp.float32),
            pltpu.SemaphoreType.DMA,
            pltpu.SemaphoreType.DMA,
        ],
    )
    def k(table_hbm, idx_hbm, y_hbm, idx_all, rows, gsem, wsem):
        wid = lax.axis_index("s") * 2 + lax.axis_index("c")
        base = wid * ROWS_PER_W
        pltpu.sync_copy(idx_hbm.at[pl.ds(base, ROWS_PER_W)], idx_all)

        def gather(j, b):
            pltpu.async_copy(
                table_hbm.at[idx_all.at[pl.ds(j * CHUNK, CHUNK)]],
                rows.at[b], gsem)

        def gwait(b):
            pltpu.make_async_copy(
                table_hbm.at[pl.ds(0, CHUNK)], rows.at[b], gsem).wait()

        def wback(j, b):
            pltpu.async_copy(
                rows.at[b], y_hbm.at[pl.ds(base + j * CHUNK, CHUNK)], wsem)

        def wwait(b):
            pltpu.make_async_copy(
                table_hbm.at[pl.ds(0, CHUNK)], rows.at[b], wsem).wait()

        for b in range(NBUF):
            gather(b, b)

        def body(j0, carry):
            for b in range(NBUF):
                gwait(b)
                wback(j0 * NBUF + b, b)
            for b in range(NBUF):
                nxt = (j0 + 1) * NBUF + b

                @pl.when(nxt < N_CHUNKS)
                def _():
                    wwait(b)
                    gather(nxt, b)

            return carry

        lax.fori_loop(0, N_GROUPS, body, 0)
        for b in range(NBUF):
            wwait(b)

    return k(table, idx_flat)


def _stage3_body(wk_ref, wsum_ref, b_ref, x_ref, y_ref, out_ref):
    acc = lax.dot_general(
        wsum_ref[...], x_ref[...], (((1,), (0,)), ((), ())),
        preferred_element_type=jnp.float32, precision=lax.Precision.DEFAULT)
    for k in range(N_SEL):
        acc = acc - lax.dot_general(
            wk_ref[k], y_ref[k], (((1,), (1,)), ((), ())),
            preferred_element_type=jnp.float32, precision=lax.Precision.DEFAULT)
    out_ref[...] = acc + b_ref[...]


def _stage3(wk, wsum, b2, x_cm, y):
    return pl.pallas_call(
        _stage3_body,
        grid=(N_TILES_3,),
        in_specs=[
            pl.BlockSpec((N_SEL, C, CP), lambda i: (0, 0, 0)),
            pl.BlockSpec((C, C), lambda i: (0, 0)),
            pl.BlockSpec((C, 1), lambda i: (0, 0)),
            pl.BlockSpec((C, TP), lambda i: (0, i)),
            pl.BlockSpec((N_SEL, TP, CP), lambda i: (0, i, 0)),
        ],
        out_specs=pl.BlockSpec((C, TP), lambda i: (0, i)),
        out_shape=jax.ShapeDtypeStruct((C, PIX_H), jnp.float32),
    )(wk, wsum, b2, x_cm, y)


def kernel(x, W, b_conv):
    xsq = x[0]
    x_pad4 = jnp.pad(xsq, ((0, 0), (PAD_T, PAD_T + 4), (PAD_L, PAD_L)))
    x_pad = x_pad4[:, :HP, :]
    table = jnp.pad(
        jnp.transpose(x_pad.reshape(C, HP * WP), (1, 0)), ((0, 0), (0, CP - C)))
    w2 = W.reshape(C, C, N_SEL)
    wk = jnp.pad(jnp.transpose(w2, (2, 0, 1)), ((0, 0), (0, 0), (0, CP - C)))
    wsum = jnp.sum(w2, axis=2)
    b2 = b_conv.reshape(C, 1)
    outs = []
    idxs = [_stage1(x_pad4, h * H_HALF) for h in range(HALF)]
    ys = [_sc_gather(table, idxs[h].reshape(-1)) for h in range(HALF)]
    for h in range(HALF):
        x_cm = xsq[:, h * H_HALF:(h + 1) * H_HALF, :].reshape(C, PIX_H)
        outs.append(
            _stage3(wk, wsum, b2, x_cm, ys[h].reshape(N_SEL, PIX_H, CP)))
    out = jnp.concatenate(outs, axis=1)
    return out.reshape(1, C, H, W_DIM)
